# baseline (speedup 1.0000x reference)
import functools

import jax
import jax.numpy as jnp
from jax.experimental import pallas as pl
from jax.experimental.pallas import tpu as pltpu

N_SLOTS = 8
D_INPUT = 384
D_SLOT = 128
MLP_HIDDEN = 256
N_ITERS = 3
EPS = 1e-8
THRESHOLD = 0.9
SCALE = D_SLOT ** -0.5
LN_EPS = 1e-5
BB = 2
NCH = 8


def _ln(x, g, b):
    m = jnp.mean(x, axis=-1, keepdims=True)
    v = jnp.mean((x - m) ** 2, axis=-1, keepdims=True)
    return (x - m) * jax.lax.rsqrt(v + LN_EPS) * g + b


def _dot_t(a, b):
    return jax.lax.dot_general(a, b, (((1,), (1,)), ((), ())),
                               preferred_element_type=jnp.float32)


def _dot(a, b):
    return jnp.dot(a, b, preferred_element_type=jnp.float32)


def _slot_encoder_kernel(feat_ref, wkv, cw, cb, wq, ln_s_g, ln_s_b,
                         w_ih, w_hh, b_ih, b_hh, ln_m_g, ln_m_b,
                         w1, b1, w2, b2, mu,
                         merged_ref, attn_ref, mm_ref, raw_ref,
                         yb_scr, rs_scr, mrs_scr):
    i = pl.program_id(0)
    par = jax.lax.rem(i, 2)
    prev = 1 - par
    cwk, cwv = cw[:, :D_SLOT], cw[:, D_SLOT:]
    cbk, cbv = cb[:, :D_SLOT], cb[:, D_SLOT:]
    n_full = feat_ref.shape[1]
    ch = n_full // NCH

    p1_state = {"k": 0}
    mr_parts = [[] for _ in range(BB)]
    e2_parts = [[] for _ in range(BB)]

    o384 = jnp.full((1, D_INPUT), 1.0 / D_INPUT, jnp.bfloat16)

    def p1_emit_one():
        k = p1_state["k"]
        if k >= BB * NCH:
            return
        p1_state["k"] = k + 1
        j, c = divmod(k, NCH)
        xc = feat_ref[j, pl.ds(c * ch, ch), :]
        xb = xc.astype(jnp.bfloat16)
        x2b = (xc * xc).astype(jnp.bfloat16)
        ytc = _dot_t(wkv[...], xb)
        mr_parts[j].append(ytc[2 * D_SLOT:2 * D_SLOT + 1, :])
        e2_parts[j].append(_dot_t(o384, x2b))
        yb_scr[par, j, :, pl.ds(c * ch, ch)] = (
            ytc[:2 * D_SLOT, :].astype(jnp.bfloat16))
        if c == NCH - 1:
            mr = jnp.concatenate(mr_parts[j], axis=1)
            e2r = jnp.concatenate(e2_parts[j], axis=1)
            rsr = jax.lax.rsqrt(e2r - mr * mr + LN_EPS)
            rs_scr[par, j] = rsr
            mrs_scr[par, j] = mr * rsr

    yk_load = lambda j: yb_scr[prev, j, pl.ds(0, D_SLOT), :]
    yv_load = lambda j: yb_scr[prev, j, pl.ds(D_SLOT, D_SLOT), :]
    rs_rows, mrs_rows = [], []
    for j in range(BB):
        rs_rows.append(rs_scr[prev, j])
        mrs_rows.append(mrs_scr[prev, j])

    S = BB * N_SLOTS
    slots = jnp.broadcast_to(mu[...], (S, D_SLOT))
    attns = None
    for _ in range(N_ITERS):
        s_prev = slots
        s = _ln(slots, ln_s_g[...], ln_s_b[...])
        q = _dot(s, wq[...])
        qb = q.astype(jnp.bfloat16)
        attns, upds = [], []
        for j in range(BB):
            qj = q[j * N_SLOTS:(j + 1) * N_SLOTS]
            lraw = _dot(qb[j * N_SLOTS:(j + 1) * N_SLOTS], yk_load(j))
            ak = jnp.sum(qj * cwk, axis=1, keepdims=True)
            ck = jnp.sum(qj * cbk, axis=1, keepdims=True)
            logits = ((rs_rows[j] * SCALE) * lraw
                      - ak * (mrs_rows[j] * SCALE) + ck * SCALE)
            e = jnp.exp(logits)
            attn = e / jnp.sum(e, axis=0, keepdims=True)
            ssum = jnp.sum(attn, axis=1, keepdims=True)
            attn_n = attn / (ssum + EPS)
            attns.append(attn)
            asc = attn_n * rs_rows[j]
            t1 = jnp.sum(attn_n * mrs_rows[j], axis=1, keepdims=True)
            t0 = ssum / (ssum + EPS)
            upds.append(_dot_t(asc.astype(jnp.bfloat16), yv_load(j))
                        - t1 * cwv + t0 * cbv)
            p1_emit_one()
        updates = jnp.concatenate(upds, axis=0)
        gx = _dot_t(updates, w_ih[...]) + b_ih[...]
        gh = _dot_t(s_prev, w_hh[...]) + b_hh[...]
        r = jax.nn.sigmoid(gx[:, :D_SLOT] + gh[:, :D_SLOT])
        z = jax.nn.sigmoid(gx[:, D_SLOT:2 * D_SLOT] + gh[:, D_SLOT:2 * D_SLOT])
        n = jnp.tanh(gx[:, 2 * D_SLOT:] + r * gh[:, 2 * D_SLOT:])
        slots = (1.0 - z) * n + z * s_prev
        h = _ln(slots, ln_m_g[...], ln_m_b[...])
        slots = slots + _dot(jax.nn.silu(_dot(h, w1[...]) + b1[...]),
                             w2[...]) + b2[...]
        p1_emit_one()

    norm = jnp.sqrt(jnp.sum(slots * slots, axis=-1, keepdims=True))
    sn = slots / jnp.maximum(norm, 1e-12)
    iota_r = jax.lax.broadcasted_iota(jnp.int32, (N_SLOTS, N_SLOTS), 0)
    iota_c = jax.lax.broadcasted_iota(jnp.int32, (N_SLOTS, N_SLOTS), 1)
    eye2 = 2.0 * jnp.where(iota_r == iota_c, 1.0, 0.0)
    fid = iota_r * N_SLOTS + iota_c
    lane = jax.lax.broadcasted_iota(jnp.int32, (1, N_SLOTS), 1)
    sims, mts = [], []
    for j in range(BB):
        snj = sn[j * N_SLOTS:(j + 1) * N_SLOTS]
        sims.append(_dot_t(snj, snj) - eye2)
        mts.append(lane)
    p1_emit_one()
    for _ in range(N_SLOTS):
        for j in range(BB):
            sim = sims[j]
            mx_sim = jnp.max(jnp.max(sim, axis=1, keepdims=True),
                             axis=0, keepdims=True)
            cand = jnp.where(sim == mx_sim, fid, N_SLOTS * N_SLOTS)
            idx = jnp.min(jnp.min(cand, axis=1, keepdims=True),
                          axis=0, keepdims=True)
            row = idx // N_SLOTS
            col = idx - row * N_SLOTS
            cond = mx_sim > THRESHOLD
            src = jnp.maximum(row, col)
            tgt = jnp.minimum(row, col)
            mts[j] = jnp.where(jnp.logical_and(cond, lane == src),
                               tgt, mts[j])
            hit = jnp.logical_or(iota_r == src, iota_c == src)
            sims[j] = jnp.where(jnp.logical_and(cond, hit), -2.0, sim)
        p1_emit_one()

    t_iota = jax.lax.broadcasted_iota(jnp.int32, (N_SLOTS, N_SLOTS), 0)
    for j in range(BB):
        sl = slots[j * N_SLOTS:(j + 1) * N_SLOTS]
        oh = (mts[j] == t_iota).astype(jnp.float32)
        summed = _dot(oh, sl)
        counts = jnp.sum(oh, axis=1, keepdims=True)
        merged_ref[j] = summed / jnp.maximum(counts, 1.0)
        mm_ref[j] = mts[j]
        raw_ref[j] = sl
        attn_ref[j] = attns[j]
        p1_emit_one()

    for _ in range(BB * NCH):
        p1_emit_one()


@functools.partial(jax.jit, static_argnames=("interpret",))
def kernel(features, ln_in_g, ln_in_b, Wk, Wv, Wq, ln_s_g, ln_s_b, W_ih,
           W_hh, b_ih, b_hh, ln_m_g, ln_m_b, W1, b1, W2, b2, slot_mu,
           interpret=False):
    B, N, _ = features.shape
    G = B // BB
    Wkv_raw = jnp.concatenate([Wk, Wv], axis=1)
    Wkv = Wkv_raw * ln_in_g[:, None]
    cw = jnp.sum(Wkv, axis=0).reshape(1, -1)
    cb = (ln_in_b @ Wkv_raw).reshape(1, -1)
    row = lambda a: a.reshape(1, -1)
    full = lambda a: pl.BlockSpec(a.shape, lambda b: (0,) * a.ndim)
    WkvTaug = jnp.concatenate(
        [Wkv.T, jnp.full((1, D_INPUT), 1.0 / D_INPUT, Wkv.dtype)], axis=0)
    ins = [features, WkvTaug.astype(jnp.bfloat16), cw, cb, Wq, row(ln_s_g),
           row(ln_s_b), W_ih, W_hh, row(b_ih), row(b_hh), row(ln_m_g),
           row(ln_m_b), W1, row(b1), W2, row(b2), slot_mu.reshape(1, D_SLOT)]
    in_specs = [pl.BlockSpec((BB, N, D_INPUT),
                             lambda i: (jnp.minimum(i, G - 1), 0, 0))]
    in_specs += [full(a) for a in ins[1:]]
    lag = lambda i: (jnp.maximum(i - 1, 0), 0, 0)
    out_shape = (
        jax.ShapeDtypeStruct((B, N_SLOTS, D_SLOT), jnp.float32),
        jax.ShapeDtypeStruct((B, N_SLOTS, N), jnp.float32),
        jax.ShapeDtypeStruct((B, 1, N_SLOTS), jnp.int32),
        jax.ShapeDtypeStruct((B, N_SLOTS, D_SLOT), jnp.float32),
    )
    out_specs = (
        pl.BlockSpec((BB, N_SLOTS, D_SLOT), lag),
        pl.BlockSpec((BB, N_SLOTS, N), lag),
        pl.BlockSpec((BB, 1, N_SLOTS), lag),
        pl.BlockSpec((BB, N_SLOTS, D_SLOT), lag),
    )
    merged, attn, mm, raw = pl.pallas_call(
        _slot_encoder_kernel,
        out_shape=out_shape,
        grid=(G + 1,),
        in_specs=in_specs,
        out_specs=out_specs,
        scratch_shapes=[
            pltpu.VMEM((2, BB, 2 * D_SLOT, N), jnp.bfloat16),
            pltpu.VMEM((2, BB, 1, N), jnp.float32),
            pltpu.VMEM((2, BB, 1, N), jnp.float32),
        ],
        compiler_params=pltpu.CompilerParams(
            dimension_semantics=("arbitrary",),
            vmem_limit_bytes=56 * 1024 * 1024,
        ),
        name="slot_encoder_fused",
        interpret=interpret,
    )(*ins)
    return merged, attn, mm.reshape(B, N_SLOTS), raw


# speedup vs baseline: 1.0721x; 1.0721x over previous
import functools

import jax
import jax.numpy as jnp
from jax.experimental import pallas as pl
from jax.experimental.pallas import tpu as pltpu

N_SLOTS = 8
D_INPUT = 384
D_SLOT = 128
MLP_HIDDEN = 256
N_ITERS = 3
EPS = 1e-8
THRESHOLD = 0.9
SCALE = D_SLOT ** -0.5
LN_EPS = 1e-5
BB = 2
NCH = 8


def _ln(x, g, b):
    m = jnp.mean(x, axis=-1, keepdims=True)
    v = jnp.mean((x - m) ** 2, axis=-1, keepdims=True)
    return (x - m) * jax.lax.rsqrt(v + LN_EPS) * g + b


def _dot_t(a, b):
    return jax.lax.dot_general(a, b, (((1,), (1,)), ((), ())),
                               preferred_element_type=jnp.float32)


def _dot(a, b):
    return jnp.dot(a, b, preferred_element_type=jnp.float32)


def _slot_encoder_kernel(feat_ref, wkv, cw, cb, wq, ln_s_g, ln_s_b,
                         w_ih, w_hh, b_ih, b_hh, ln_m_g, ln_m_b,
                         w1, b1, w2, b2, mu,
                         merged_ref, attn_ref, mm_ref, raw_ref,
                         yb_scr, rs_scr, mrs_scr):
    i = pl.program_id(0)
    par = jax.lax.rem(i, 2)
    prev = 1 - par
    cwk, cwv = cw[:, :D_SLOT], cw[:, D_SLOT:]
    cbk, cbv = cb[:, :D_SLOT], cb[:, D_SLOT:]
    n_full = feat_ref.shape[1]
    ch = n_full // NCH

    p1_state = {"k": 0}
    mr_parts = [[] for _ in range(BB)]
    e2_parts = [[] for _ in range(BB)]

    def p1_emit_one():
        k = p1_state["k"]
        if k >= BB * NCH:
            return
        p1_state["k"] = k + 1
        j, c = divmod(k, NCH)
        xc = feat_ref[j, pl.ds(c * ch, ch), :]
        m = jnp.mean(xc, axis=-1, keepdims=True)
        ex2 = jnp.mean(xc * xc, axis=-1, keepdims=True)
        mr_parts[j].append(jnp.reshape(m, (1, -1)))
        e2_parts[j].append(jnp.reshape(ex2, (1, -1)))
        yc = _dot(xc.astype(jnp.bfloat16), wkv[...])
        yb_scr[par, j, pl.ds(c * ch, ch), :] = yc.astype(jnp.bfloat16)
        if c == NCH - 1:
            mr = jnp.concatenate(mr_parts[j], axis=1)
            e2r = jnp.concatenate(e2_parts[j], axis=1)
            rsr = jax.lax.rsqrt(e2r - mr * mr + LN_EPS)
            rs_scr[par, j] = rsr
            mrs_scr[par, j] = mr * rsr

    yk_load = lambda j: yb_scr[prev, j, :, pl.ds(0, D_SLOT)]
    yv_load = lambda j: yb_scr[prev, j, :, pl.ds(D_SLOT, D_SLOT)]
    rs_rows, mrs_rows = [], []
    for j in range(BB):
        rs_rows.append(rs_scr[prev, j])
        mrs_rows.append(mrs_scr[prev, j])

    S = BB * N_SLOTS
    slots = jnp.broadcast_to(mu[...], (S, D_SLOT))
    attns = None
    for _ in range(N_ITERS):
        s_prev = slots
        s = _ln(slots, ln_s_g[...], ln_s_b[...])
        q = _dot(s, wq[...])
        qb = q.astype(jnp.bfloat16)
        attns, upds = [], []
        for j in range(BB):
            qj = q[j * N_SLOTS:(j + 1) * N_SLOTS]
            lraw = _dot_t(qb[j * N_SLOTS:(j + 1) * N_SLOTS], yk_load(j))
            ak = jnp.sum(qj * cwk, axis=1, keepdims=True)
            ck = jnp.sum(qj * cbk, axis=1, keepdims=True)
            logits = ((rs_rows[j] * SCALE) * lraw
                      - ak * (mrs_rows[j] * SCALE) + ck * SCALE)
            e = jnp.exp(logits)
            attn = e / jnp.sum(e, axis=0, keepdims=True)
            ssum = jnp.sum(attn, axis=1, keepdims=True)
            attn_n = attn / (ssum + EPS)
            attns.append(attn)
            asc = attn_n * rs_rows[j]
            t1 = jnp.sum(attn_n * mrs_rows[j], axis=1, keepdims=True)
            t0 = ssum / (ssum + EPS)
            upds.append(_dot(asc.astype(jnp.bfloat16), yv_load(j))
                        - t1 * cwv + t0 * cbv)
            p1_emit_one()
        updates = jnp.concatenate(upds, axis=0)
        gx = _dot_t(updates, w_ih[...]) + b_ih[...]
        gh = _dot_t(s_prev, w_hh[...]) + b_hh[...]
        r = jax.nn.sigmoid(gx[:, :D_SLOT] + gh[:, :D_SLOT])
        z = jax.nn.sigmoid(gx[:, D_SLOT:2 * D_SLOT] + gh[:, D_SLOT:2 * D_SLOT])
        n = jnp.tanh(gx[:, 2 * D_SLOT:] + r * gh[:, 2 * D_SLOT:])
        slots = (1.0 - z) * n + z * s_prev
        h = _ln(slots, ln_m_g[...], ln_m_b[...])
        slots = slots + _dot(jax.nn.silu(_dot(h, w1[...]) + b1[...]),
                             w2[...]) + b2[...]
        p1_emit_one()

    norm = jnp.sqrt(jnp.sum(slots * slots, axis=-1, keepdims=True))
    sn = slots / jnp.maximum(norm, 1e-12)
    iota_r = jax.lax.broadcasted_iota(jnp.int32, (N_SLOTS, N_SLOTS), 0)
    iota_c = jax.lax.broadcasted_iota(jnp.int32, (N_SLOTS, N_SLOTS), 1)
    eye2 = 2.0 * jnp.where(iota_r == iota_c, 1.0, 0.0)
    fid = iota_r * N_SLOTS + iota_c
    lane = jax.lax.broadcasted_iota(jnp.int32, (1, N_SLOTS), 1)
    sims, mts = [], []
    for j in range(BB):
        snj = sn[j * N_SLOTS:(j + 1) * N_SLOTS]
        sims.append(_dot_t(snj, snj) - eye2)
        mts.append(lane)
    p1_emit_one()
    for _ in range(N_SLOTS):
        for j in range(BB):
            sim = sims[j]
            mx_sim = jnp.max(jnp.max(sim, axis=1, keepdims=True),
                             axis=0, keepdims=True)
            cand = jnp.where(sim == mx_sim, fid, N_SLOTS * N_SLOTS)
            idx = jnp.min(jnp.min(cand, axis=1, keepdims=True),
                          axis=0, keepdims=True)
            row = idx // N_SLOTS
            col = idx - row * N_SLOTS
            cond = mx_sim > THRESHOLD
            src = jnp.maximum(row, col)
            tgt = jnp.minimum(row, col)
            mts[j] = jnp.where(jnp.logical_and(cond, lane == src),
                               tgt, mts[j])
            hit = jnp.logical_or(iota_r == src, iota_c == src)
            sims[j] = jnp.where(jnp.logical_and(cond, hit), -2.0, sim)
        p1_emit_one()

    t_iota = jax.lax.broadcasted_iota(jnp.int32, (N_SLOTS, N_SLOTS), 0)
    for j in range(BB):
        sl = slots[j * N_SLOTS:(j + 1) * N_SLOTS]
        oh = (mts[j] == t_iota).astype(jnp.float32)
        summed = _dot(oh, sl)
        counts = jnp.sum(oh, axis=1, keepdims=True)
        merged_ref[j] = summed / jnp.maximum(counts, 1.0)
        mm_ref[j] = mts[j]
        raw_ref[j] = sl
        attn_ref[j] = attns[j]
        p1_emit_one()

    for _ in range(BB * NCH):
        p1_emit_one()


@functools.partial(jax.jit, static_argnames=("interpret",))
def kernel(features, ln_in_g, ln_in_b, Wk, Wv, Wq, ln_s_g, ln_s_b, W_ih,
           W_hh, b_ih, b_hh, ln_m_g, ln_m_b, W1, b1, W2, b2, slot_mu,
           interpret=False):
    B, N, _ = features.shape
    G = B // BB
    Wkv_raw = jnp.concatenate([Wk, Wv], axis=1)
    Wkv = Wkv_raw * ln_in_g[:, None]
    cw = jnp.sum(Wkv, axis=0).reshape(1, -1)
    cb = (ln_in_b @ Wkv_raw).reshape(1, -1)
    row = lambda a: a.reshape(1, -1)
    full = lambda a: pl.BlockSpec(a.shape, lambda b: (0,) * a.ndim)
    ins = [features, Wkv.astype(jnp.bfloat16), cw, cb, Wq, row(ln_s_g),
           row(ln_s_b), W_ih, W_hh, row(b_ih), row(b_hh), row(ln_m_g),
           row(ln_m_b), W1, row(b1), W2, row(b2), slot_mu.reshape(1, D_SLOT)]
    in_specs = [pl.BlockSpec((BB, N, D_INPUT),
                             lambda i: (jnp.minimum(i, G - 1), 0, 0))]
    in_specs += [full(a) for a in ins[1:]]
    lag = lambda i: (jnp.maximum(i - 1, 0), 0, 0)
    out_shape = (
        jax.ShapeDtypeStruct((B, N_SLOTS, D_SLOT), jnp.float32),
        jax.ShapeDtypeStruct((B, N_SLOTS, N), jnp.float32),
        jax.ShapeDtypeStruct((B, 1, N_SLOTS), jnp.int32),
        jax.ShapeDtypeStruct((B, N_SLOTS, D_SLOT), jnp.float32),
    )
    out_specs = (
        pl.BlockSpec((BB, N_SLOTS, D_SLOT), lag),
        pl.BlockSpec((BB, N_SLOTS, N), lag),
        pl.BlockSpec((BB, 1, N_SLOTS), lag),
        pl.BlockSpec((BB, N_SLOTS, D_SLOT), lag),
    )
    merged, attn, mm, raw = pl.pallas_call(
        _slot_encoder_kernel,
        out_shape=out_shape,
        grid=(G + 1,),
        in_specs=in_specs,
        out_specs=out_specs,
        scratch_shapes=[
            pltpu.VMEM((2, BB, N, 2 * D_SLOT), jnp.bfloat16),
            pltpu.VMEM((2, BB, 1, N), jnp.float32),
            pltpu.VMEM((2, BB, 1, N), jnp.float32),
        ],
        compiler_params=pltpu.CompilerParams(
            dimension_semantics=("arbitrary",),
            vmem_limit_bytes=56 * 1024 * 1024,
        ),
        name="slot_encoder_fused",
        interpret=interpret,
    )(*ins)
    return merged, attn, mm.reshape(B, N_SLOTS), raw


# speedup vs baseline: 1.0772x; 1.0048x over previous
import jax
import jax.numpy as jnp
from jax.experimental import pallas as pl
from jax.experimental.pallas import tpu as pltpu

N_SLOTS = 8
D_INPUT = 384
D_SLOT = 128
MLP_HIDDEN = 256
N_ITERS = 3
EPS = 1e-8
THRESHOLD = 0.9
SCALE = D_SLOT ** -0.5
LN_EPS = 1e-5
BB = 2
NCH = 8


def _ln(x, g, b):
    m = jnp.mean(x, axis=-1, keepdims=True)
    v = jnp.mean((x - m) ** 2, axis=-1, keepdims=True)
    return (x - m) * jax.lax.rsqrt(v + LN_EPS) * g + b


def _dot_t(a, b):
    return jax.lax.dot_general(a, b, (((1,), (1,)), ((), ())),
                               preferred_element_type=jnp.float32)


def _dot(a, b):
    return jnp.dot(a, b, preferred_element_type=jnp.float32)


def _slot_encoder_kernel(feat_ref, wkv, cw, cb, wq, ln_s_g, ln_s_b,
                         w_ih, w_hh, b_ih, b_hh, ln_m_g, ln_m_b,
                         w1, b1, w2, b2, mu,
                         merged_ref, attn_ref, mm_ref, raw_ref,
                         yb_scr, rs_scr, mrs_scr):
    i = pl.program_id(0)
    par = jax.lax.rem(i, 2)
    prev = 1 - par
    cwk, cwv = cw[:, :D_SLOT], cw[:, D_SLOT:]
    cbk, cbv = cb[:, :D_SLOT], cb[:, D_SLOT:]
    n_full = feat_ref.shape[1]
    ch = n_full // NCH

    p1_state = {"k": 0}
    mr_parts = [[] for _ in range(BB)]
    e2_parts = [[] for _ in range(BB)]

    def p1_emit_one():
        k = p1_state["k"]
        if k >= BB * NCH:
            return
        p1_state["k"] = k + 1
        j, c = divmod(k, NCH)
        xc = feat_ref[j, pl.ds(c * ch, ch), :]
        m = jnp.mean(xc, axis=-1, keepdims=True)
        ex2 = jnp.mean(xc * xc, axis=-1, keepdims=True)
        mr_parts[j].append(jnp.reshape(m, (1, -1)))
        e2_parts[j].append(jnp.reshape(ex2, (1, -1)))
        yc = _dot(xc.astype(jnp.bfloat16), wkv[...])
        yb_scr[par, j, pl.ds(c * ch, ch), :] = yc.astype(jnp.bfloat16)
        if c == NCH - 1:
            mr = jnp.concatenate(mr_parts[j], axis=1)
            e2r = jnp.concatenate(e2_parts[j], axis=1)
            rsr = jax.lax.rsqrt(e2r - mr * mr + LN_EPS)
            rs_scr[par, j] = rsr
            mrs_scr[par, j] = mr * rsr

    yk_load = lambda j: yb_scr[prev, j, :, pl.ds(0, D_SLOT)]
    yv_load = lambda j: yb_scr[prev, j, :, pl.ds(D_SLOT, D_SLOT)]
    rs_rows, mrs_rows = [], []
    for j in range(BB):
        rs_rows.append(rs_scr[prev, j])
        mrs_rows.append(mrs_scr[prev, j])

    S = BB * N_SLOTS
    slots = jnp.broadcast_to(mu[...], (S, D_SLOT))
    attns = None
    for _ in range(N_ITERS):
        s_prev = slots
        s = _ln(slots, ln_s_g[...], ln_s_b[...])
        q = _dot(s, wq[...])
        qb = q.astype(jnp.bfloat16)
        attns, upds = [], []
        for j in range(BB):
            qj = q[j * N_SLOTS:(j + 1) * N_SLOTS]
            lraw = _dot_t(qb[j * N_SLOTS:(j + 1) * N_SLOTS], yk_load(j))
            ak = jnp.sum(qj * cwk, axis=1, keepdims=True)
            ck = jnp.sum(qj * cbk, axis=1, keepdims=True)
            logits = ((rs_rows[j] * SCALE) * lraw
                      - ak * (mrs_rows[j] * SCALE) + ck * SCALE)
            e = jnp.exp(logits)
            attn = e / jnp.sum(e, axis=0, keepdims=True)
            ssum = jnp.sum(attn, axis=1, keepdims=True)
            attn_n = attn / (ssum + EPS)
            attns.append(attn)
            asc = attn_n * rs_rows[j]
            t1 = jnp.sum(attn_n * mrs_rows[j], axis=1, keepdims=True)
            t0 = ssum / (ssum + EPS)
            upds.append(_dot(asc.astype(jnp.bfloat16), yv_load(j))
                        - t1 * cwv + t0 * cbv)
            p1_emit_one()
        updates = jnp.concatenate(upds, axis=0)
        gx = _dot_t(updates, w_ih[...]) + b_ih[...]
        gh = _dot_t(s_prev, w_hh[...]) + b_hh[...]
        r = jax.nn.sigmoid(gx[:, :D_SLOT] + gh[:, :D_SLOT])
        z = jax.nn.sigmoid(gx[:, D_SLOT:2 * D_SLOT] + gh[:, D_SLOT:2 * D_SLOT])
        n = jnp.tanh(gx[:, 2 * D_SLOT:] + r * gh[:, 2 * D_SLOT:])
        slots = (1.0 - z) * n + z * s_prev
        h = _ln(slots, ln_m_g[...], ln_m_b[...])
        slots = slots + _dot(jax.nn.silu(_dot(h, w1[...]) + b1[...]),
                             w2[...]) + b2[...]
        p1_emit_one()

    norm = jnp.sqrt(jnp.sum(slots * slots, axis=-1, keepdims=True))
    sn = slots / jnp.maximum(norm, 1e-12)
    iota_r = jax.lax.broadcasted_iota(jnp.int32, (N_SLOTS, N_SLOTS), 0)
    iota_c = jax.lax.broadcasted_iota(jnp.int32, (N_SLOTS, N_SLOTS), 1)
    eye2 = 2.0 * jnp.where(iota_r == iota_c, 1.0, 0.0)
    fid = iota_r * N_SLOTS + iota_c
    lane = jax.lax.broadcasted_iota(jnp.int32, (1, N_SLOTS), 1)
    sims, mts = [], []
    for j in range(BB):
        snj = sn[j * N_SLOTS:(j + 1) * N_SLOTS]
        sims.append(_dot_t(snj, snj) - eye2)
        mts.append(lane)
    p1_emit_one()
    for _ in range(N_SLOTS):
        for j in range(BB):
            sim = sims[j]
            mx_sim = jnp.max(jnp.max(sim, axis=1, keepdims=True),
                             axis=0, keepdims=True)
            cand = jnp.where(sim == mx_sim, fid, N_SLOTS * N_SLOTS)
            idx = jnp.min(jnp.min(cand, axis=1, keepdims=True),
                          axis=0, keepdims=True)
            row = idx // N_SLOTS
            col = idx - row * N_SLOTS
            cond = mx_sim > THRESHOLD
            src = jnp.maximum(row, col)
            tgt = jnp.minimum(row, col)
            mts[j] = jnp.where(jnp.logical_and(cond, lane == src),
                               tgt, mts[j])
            hit = jnp.logical_or(iota_r == src, iota_c == src)
            sims[j] = jnp.where(jnp.logical_and(cond, hit), -2.0, sim)
        p1_emit_one()

    t_iota = jax.lax.broadcasted_iota(jnp.int32, (N_SLOTS, N_SLOTS), 0)
    for j in range(BB):
        sl = slots[j * N_SLOTS:(j + 1) * N_SLOTS]
        oh = (mts[j] == t_iota).astype(jnp.float32)
        summed = _dot(oh, sl)
        counts = jnp.sum(oh, axis=1, keepdims=True)
        merged_ref[j] = summed / jnp.maximum(counts, 1.0)
        mm_ref[j] = mts[j]
        raw_ref[j] = sl
        attn_ref[j] = attns[j]
        p1_emit_one()

    for _ in range(BB * NCH):
        p1_emit_one()


@jax.jit
def kernel(features, ln_in_g, ln_in_b, Wk, Wv, Wq, ln_s_g, ln_s_b, W_ih,
           W_hh, b_ih, b_hh, ln_m_g, ln_m_b, W1, b1, W2, b2, slot_mu):
    B, N, _ = features.shape
    G = B // BB
    Wkv_raw = jnp.concatenate([Wk, Wv], axis=1)
    Wkv = Wkv_raw * ln_in_g[:, None]
    cw = jnp.sum(Wkv, axis=0).reshape(1, -1)
    cb = (ln_in_b @ Wkv_raw).reshape(1, -1)
    row = lambda a: a.reshape(1, -1)
    full = lambda a: pl.BlockSpec(a.shape, lambda b: (0,) * a.ndim)
    ins = [features, Wkv.astype(jnp.bfloat16), cw, cb, Wq, row(ln_s_g),
           row(ln_s_b), W_ih, W_hh, row(b_ih), row(b_hh), row(ln_m_g),
           row(ln_m_b), W1, row(b1), W2, row(b2), slot_mu.reshape(1, D_SLOT)]
    in_specs = [pl.BlockSpec((BB, N, D_INPUT),
                             lambda i: (jnp.minimum(i, G - 1), 0, 0))]
    in_specs += [full(a) for a in ins[1:]]
    lag = lambda i: (jnp.maximum(i - 1, 0), 0, 0)
    out_shape = (
        jax.ShapeDtypeStruct((B, N_SLOTS, D_SLOT), jnp.float32),
        jax.ShapeDtypeStruct((B, N_SLOTS, N), jnp.float32),
        jax.ShapeDtypeStruct((B, 1, N_SLOTS), jnp.int32),
        jax.ShapeDtypeStruct((B, N_SLOTS, D_SLOT), jnp.float32),
    )
    out_specs = (
        pl.BlockSpec((BB, N_SLOTS, D_SLOT), lag),
        pl.BlockSpec((BB, N_SLOTS, N), lag),
        pl.BlockSpec((BB, 1, N_SLOTS), lag),
        pl.BlockSpec((BB, N_SLOTS, D_SLOT), lag),
    )
    merged, attn, mm, raw = pl.pallas_call(
        _slot_encoder_kernel,
        out_shape=out_shape,
        grid=(G + 1,),
        in_specs=in_specs,
        out_specs=out_specs,
        scratch_shapes=[
            pltpu.VMEM((2, BB, N, 2 * D_SLOT), jnp.bfloat16),
            pltpu.VMEM((2, BB, 1, N), jnp.float32),
            pltpu.VMEM((2, BB, 1, N), jnp.float32),
        ],
        compiler_params=pltpu.CompilerParams(
            dimension_semantics=("arbitrary",),
            vmem_limit_bytes=56 * 1024 * 1024,
        ),
        name="slot_encoder_fused",
    )(*ins)
    return merged, attn, mm.reshape(B, N_SLOTS), raw
